# Initial kernel scaffold; baseline (speedup 1.0000x reference)
import jax
import jax.numpy as jnp
from jax.experimental import pallas as pl


def kernel(x, edge_index, W0, as0, ad0, b0, W1, as1, ad1, b1, W2, as2, ad2, b2):
    raise NotImplementedError("write your pallas kernel here")


# speedup vs baseline: 61.1624x; 61.1624x over previous
import functools

import jax
import jax.numpy as jnp
from jax import lax
from jax.experimental import pallas as pl
from jax.experimental.pallas import tpu as pltpu
from jax.experimental.pallas import tpu_sc as plsc

_G = 10
_NG0 = 1000
_D = 128
_EPG = 32000
_CPAD = 1024


def _build_counts_xla(edge_index):
    src = edge_index[0]
    dst = edge_index[1]
    flat = dst * _CPAD + (src % _NG0)
    cnt = jnp.zeros((_G * _NG0 * _CPAD,), jnp.float32)
    cnt = cnt.at[flat].add(1.0)
    return cnt.reshape(_G, _NG0, _CPAD)


def _make_layer(n_g, first, last):
    n_half = n_g // 2
    n_total = n_g * _G

    def body(*refs):
        if first:
            (hin_ref, c_ref, w_ref, as_ref, ad_ref, b_ref,
             hout_ref, cout_ref, sum_ref, sq_ref) = refs
        elif last:
            (hin_ref, c_ref, w_ref, as_ref, ad_ref, b_ref, bnsum_ref,
             bnsq_ref, hout_ref, sum_ref, sq_ref) = refs
        else:
            (hin_ref, c_ref, w_ref, as_ref, ad_ref, b_ref, bnsum_ref,
             bnsq_ref, hout_ref, cout_ref, sum_ref, sq_ref) = refs

        hin = hin_ref[0]
        if not first:
            tot = jnp.sum(bnsum_ref[...], axis=0)
            totsq = jnp.sum(bnsq_ref[...], axis=0)
            mu = tot / n_total
            var = totsq / n_total - mu * mu
            hin = (hin - mu) * lax.rsqrt(var + 1e-5)

        h = jnp.dot(hin, w_ref[...], preferred_element_type=jnp.float32)
        avd = jnp.dot(h, ad_ref[...], preferred_element_type=jnp.float32)
        avs_row = lax.dot_general(as_ref[...], h, (((1,), (1,)), ((), ())),
                                  preferred_element_type=jnp.float32)
        e = avd + avs_row
        e = jnp.where(e >= 0, e, 0.2 * e)

        c = c_ref[0]
        if first:
            c = c[:, :n_g]
        ii = lax.broadcasted_iota(jnp.int32, (n_g, n_g), 0)
        jj = lax.broadcasted_iota(jnp.int32, (n_g, n_g), 1)
        cplus = c + jnp.where(ii == jj, 1.0, 0.0)
        em = jnp.where(cplus > 0, e, -1e9)
        m = jnp.max(em, axis=1, keepdims=True)
        wt = cplus * jnp.exp(em - m)
        denom = jnp.sum(wt, axis=1, keepdims=True) + 1e-16
        out = jnp.dot(wt, h, preferred_element_type=jnp.float32) / denom
        hrelu = jnp.maximum(out + b_ref[...], 0.0)
        hp = jnp.max(hrelu.reshape(n_half, 2, _D), axis=1)
        hout_ref[0] = hp
        sum_ref[0] = jnp.sum(hp, axis=0, keepdims=True)
        sq_ref[0] = jnp.sum(hp * hp, axis=0, keepdims=True)

        if not last:
            cb = jnp.where(c > 0, 1.0, 0.0)
            i2 = lax.broadcasted_iota(jnp.int32, (n_half, n_g), 0)
            j2 = lax.broadcasted_iota(jnp.int32, (n_half, n_g), 1)
            pt = jnp.where(j2 // 2 == i2, 1.0, 0.0)
            i3 = lax.broadcasted_iota(jnp.int32, (n_g, n_half), 0)
            j3 = lax.broadcasted_iota(jnp.int32, (n_g, n_half), 1)
            p = jnp.where(i3 // 2 == j3, 1.0, 0.0)
            s2 = jnp.dot(jnp.dot(pt, cb, preferred_element_type=jnp.float32),
                         p, preferred_element_type=jnp.float32)
            ih = lax.broadcasted_iota(jnp.int32, (n_half, n_half), 0)
            jh = lax.broadcasted_iota(jnp.int32, (n_half, n_half), 1)
            cout_ref[0] = jnp.where((s2 > 0.5) & (ih != jh), 1.0, 0.0)

    c_minor = _CPAD if first else n_g
    in_specs = [
        pl.BlockSpec((1, n_g, _D), lambda g: (g, 0, 0)),
        pl.BlockSpec((1, n_g, c_minor), lambda g: (g, 0, 0)),
        pl.BlockSpec((_D, _D), lambda g: (0, 0)),
        pl.BlockSpec((1, _D), lambda g: (0, 0)),
        pl.BlockSpec((_D, 1), lambda g: (0, 0)),
        pl.BlockSpec((1, _D), lambda g: (0, 0)),
    ]
    if not first:
        in_specs += [
            pl.BlockSpec((_G, 1, _D), lambda g: (0, 0, 0)),
            pl.BlockSpec((_G, 1, _D), lambda g: (0, 0, 0)),
        ]
    out_shapes = [jax.ShapeDtypeStruct((_G, n_half, _D), jnp.float32)]
    out_specs = [pl.BlockSpec((1, n_half, _D), lambda g: (g, 0, 0))]
    if not last:
        out_shapes.append(jax.ShapeDtypeStruct((_G, n_half, n_half),
                                               jnp.float32))
        out_specs.append(pl.BlockSpec((1, n_half, n_half),
                                      lambda g: (g, 0, 0)))
    out_shapes += [jax.ShapeDtypeStruct((_G, 1, _D), jnp.float32),
                   jax.ShapeDtypeStruct((_G, 1, _D), jnp.float32)]
    out_specs += [pl.BlockSpec((1, 1, _D), lambda g: (g, 0, 0)),
                  pl.BlockSpec((1, 1, _D), lambda g: (g, 0, 0))]

    return pl.pallas_call(
        body,
        grid=(_G,),
        in_specs=in_specs,
        out_specs=out_specs,
        out_shape=out_shapes,
    )


def _bn_final_body(hin_ref, bnsum_ref, bnsq_ref, out_ref):
    n_total = 125 * _G
    tot = jnp.sum(bnsum_ref[...], axis=0)
    totsq = jnp.sum(bnsq_ref[...], axis=0)
    mu = tot / n_total
    var = totsq / n_total - mu * mu
    out_ref[0] = (hin_ref[0] - mu) * lax.rsqrt(var + 1e-5)


def kernel(x, edge_index, W0, as0, ad0, b0, W1, as1, ad1, b1,
           W2, as2, ad2, b2):
    counts = _build_counts_xla(edge_index)

    h = x.reshape(_G, _NG0, _D)
    params = [(W0, as0, ad0, b0), (W1, as1, ad1, b1), (W2, as2, ad2, b2)]
    bnsum = bnsq = None
    n_g = _NG0
    c = counts
    for i in range(3):
        first = i == 0
        last = i == 2
        W, a_s, a_d, b = params[i]
        args = [h, c, W, a_s.reshape(1, _D), a_d.reshape(_D, 1),
                b.reshape(1, _D)]
        if not first:
            args += [bnsum, bnsq]
        outs = _make_layer(n_g, first, last)(*args)
        if last:
            h, bnsum, bnsq = outs
        else:
            h, c, bnsum, bnsq = outs
        n_g //= 2

    out = pl.pallas_call(
        _bn_final_body,
        grid=(_G,),
        in_specs=[
            pl.BlockSpec((1, 125, _D), lambda g: (g, 0, 0)),
            pl.BlockSpec((_G, 1, _D), lambda g: (0, 0, 0)),
            pl.BlockSpec((_G, 1, _D), lambda g: (0, 0, 0)),
        ],
        out_specs=pl.BlockSpec((1, 125, _D), lambda g: (g, 0, 0)),
        out_shape=jax.ShapeDtypeStruct((_G, 125, _D), jnp.float32),
    )(h, bnsum, bnsq)
    return out.reshape(_G, 125 * _D)


# speedup vs baseline: 239.9137x; 3.9226x over previous
import functools

import jax
import jax.numpy as jnp
from jax import lax
from jax.experimental import pallas as pl
from jax.experimental.pallas import tpu as pltpu
from jax.experimental.pallas import tpu_sc as plsc

_G = 10
_NG0 = 1000
_D = 128
_EPG = 32000
_CPAD = 1024


_EPT = _EPG // 16
_SPG = _CPAD * _CPAD
_WPT = _SPG // 16
_ZCH = 8192


def _sc_counts_body(src_hbm, dst_hbm, out_hbm, src_v, dst_v, idx_v, ones_v,
                    zero_v, shared):
    cid = lax.axis_index("c")
    sid = lax.axis_index("s")

    def zfill(i, _):
        zero_v[pl.ds(i * 16, 16)] = jnp.zeros((16,), jnp.float32)
        return 0
    lax.fori_loop(0, _ZCH // 16, zfill, 0)
    for r in range(8):
        ones_v[pl.ds(r * 16, 16)] = jnp.ones((16,), jnp.float32)

    for gi in range(_G // 2):
        g = gi * 2 + cid
        for z in range(_WPT // _ZCH):
            pltpu.sync_copy(zero_v,
                            shared.at[pl.ds(sid * _WPT + z * _ZCH, _ZCH)])
        ebase = g * _EPG + sid * _EPT
        pltpu.sync_copy(src_hbm.at[pl.ds(ebase, _EPT)], src_v)
        pltpu.sync_copy(dst_hbm.at[pl.ds(ebase, _EPT)], dst_v)
        goff = g * (_NG0 * _CPAD + _NG0)
        for j in range(128):
            r, col = j // 8, j % 8
            if j < _EPT // 16:
                s16 = src_v[pl.ds(j * 16, 16)]
                d16 = dst_v[pl.ds(j * 16, 16)]
                idx_v[r, pl.ds(col * 16, 16)] = d16 * _CPAD + s16 - goff
            else:
                idx_v[r, pl.ds(col * 16, 16)] = jnp.full((16,), _NG0 * _CPAD,
                                                         jnp.int32)
        plsc.subcore_barrier()
        for r in range(16):
            pltpu.sync_copy(ones_v, shared.at[idx_v.at[r]], add=True)
        plsc.subcore_barrier()
        gbase = g * (_NG0 * _CPAD)

        @pl.when(sid < 15)
        def _():
            pltpu.sync_copy(shared.at[pl.ds(sid * _WPT, _WPT)],
                            out_hbm.at[pl.ds(gbase + sid * _WPT, _WPT)])

        @pl.when(sid == 15)
        def _():
            nlast = (_NG0 - 15 * 64) * _CPAD
            pltpu.sync_copy(shared.at[pl.ds(15 * _WPT, nlast)],
                            out_hbm.at[pl.ds(gbase + 15 * _WPT, nlast)])
        plsc.subcore_barrier()


_sc_counts = functools.partial(
    pl.kernel,
    out_type=jax.ShapeDtypeStruct((_G * _NG0 * _CPAD,), jnp.float32),
    mesh=plsc.VectorSubcoreMesh(core_axis_name="c", subcore_axis_name="s"),
    scratch_types=[
        pltpu.VMEM((_EPT,), jnp.int32),
        pltpu.VMEM((_EPT,), jnp.int32),
        pltpu.VMEM((16, 128), jnp.int32),
        pltpu.VMEM((128,), jnp.float32),
        pltpu.VMEM((_ZCH,), jnp.float32),
        pltpu.VMEM_SHARED((_SPG,), jnp.float32),
    ],
)(_sc_counts_body)


def _build_counts(edge_index):
    return _sc_counts(edge_index[0],
                      edge_index[1]).reshape(_G, _NG0, _CPAD)


def _make_layer(n_g, first, last):
    n_half = n_g // 2
    n_total = n_g * _G

    def body(*refs):
        if first:
            (hin_ref, c_ref, w_ref, as_ref, ad_ref, b_ref,
             hout_ref, cout_ref, sum_ref, sq_ref) = refs
        elif last:
            (hin_ref, c_ref, w_ref, as_ref, ad_ref, b_ref, bnsum_ref,
             bnsq_ref, hout_ref, sum_ref, sq_ref) = refs
        else:
            (hin_ref, c_ref, w_ref, as_ref, ad_ref, b_ref, bnsum_ref,
             bnsq_ref, hout_ref, cout_ref, sum_ref, sq_ref) = refs

        hin = hin_ref[0]
        if not first:
            tot = jnp.sum(bnsum_ref[...], axis=0)
            totsq = jnp.sum(bnsq_ref[...], axis=0)
            mu = tot / n_total
            var = totsq / n_total - mu * mu
            hin = (hin - mu) * lax.rsqrt(var + 1e-5)

        h = jnp.dot(hin, w_ref[...], preferred_element_type=jnp.float32)
        avd = jnp.dot(h, ad_ref[...], preferred_element_type=jnp.float32)
        avs_row = lax.dot_general(as_ref[...], h, (((1,), (1,)), ((), ())),
                                  preferred_element_type=jnp.float32)
        e = avd + avs_row
        e = jnp.where(e >= 0, e, 0.2 * e)

        c = c_ref[0]
        if first:
            c = c[:, :n_g]
        ii = lax.broadcasted_iota(jnp.int32, (n_g, n_g), 0)
        jj = lax.broadcasted_iota(jnp.int32, (n_g, n_g), 1)
        cplus = c + jnp.where(ii == jj, 1.0, 0.0)
        em = jnp.where(cplus > 0, e, -1e9)
        m = jnp.max(em, axis=1, keepdims=True)
        wt = cplus * jnp.exp(em - m)
        denom = jnp.sum(wt, axis=1, keepdims=True) + 1e-16
        out = jnp.dot(wt, h, preferred_element_type=jnp.float32) / denom
        hrelu = jnp.maximum(out + b_ref[...], 0.0)
        hp = jnp.max(hrelu.reshape(n_half, 2, _D), axis=1)
        hout_ref[0] = hp
        sum_ref[0] = jnp.sum(hp, axis=0, keepdims=True)
        sq_ref[0] = jnp.sum(hp * hp, axis=0, keepdims=True)

        if not last:
            cb = jnp.where(c > 0, 1.0, 0.0)
            i2 = lax.broadcasted_iota(jnp.int32, (n_half, n_g), 0)
            j2 = lax.broadcasted_iota(jnp.int32, (n_half, n_g), 1)
            pt = jnp.where(j2 // 2 == i2, 1.0, 0.0)
            i3 = lax.broadcasted_iota(jnp.int32, (n_g, n_half), 0)
            j3 = lax.broadcasted_iota(jnp.int32, (n_g, n_half), 1)
            p = jnp.where(i3 // 2 == j3, 1.0, 0.0)
            s2 = jnp.dot(jnp.dot(pt, cb, preferred_element_type=jnp.float32),
                         p, preferred_element_type=jnp.float32)
            ih = lax.broadcasted_iota(jnp.int32, (n_half, n_half), 0)
            jh = lax.broadcasted_iota(jnp.int32, (n_half, n_half), 1)
            cout_ref[0] = jnp.where((s2 > 0.5) & (ih != jh), 1.0, 0.0)

    c_minor = _CPAD if first else n_g
    in_specs = [
        pl.BlockSpec((1, n_g, _D), lambda g: (g, 0, 0)),
        pl.BlockSpec((1, n_g, c_minor), lambda g: (g, 0, 0)),
        pl.BlockSpec((_D, _D), lambda g: (0, 0)),
        pl.BlockSpec((1, _D), lambda g: (0, 0)),
        pl.BlockSpec((_D, 1), lambda g: (0, 0)),
        pl.BlockSpec((1, _D), lambda g: (0, 0)),
    ]
    if not first:
        in_specs += [
            pl.BlockSpec((_G, 1, _D), lambda g: (0, 0, 0)),
            pl.BlockSpec((_G, 1, _D), lambda g: (0, 0, 0)),
        ]
    out_shapes = [jax.ShapeDtypeStruct((_G, n_half, _D), jnp.float32)]
    out_specs = [pl.BlockSpec((1, n_half, _D), lambda g: (g, 0, 0))]
    if not last:
        out_shapes.append(jax.ShapeDtypeStruct((_G, n_half, n_half),
                                               jnp.float32))
        out_specs.append(pl.BlockSpec((1, n_half, n_half),
                                      lambda g: (g, 0, 0)))
    out_shapes += [jax.ShapeDtypeStruct((_G, 1, _D), jnp.float32),
                   jax.ShapeDtypeStruct((_G, 1, _D), jnp.float32)]
    out_specs += [pl.BlockSpec((1, 1, _D), lambda g: (g, 0, 0)),
                  pl.BlockSpec((1, 1, _D), lambda g: (g, 0, 0))]

    return pl.pallas_call(
        body,
        grid=(_G,),
        in_specs=in_specs,
        out_specs=out_specs,
        out_shape=out_shapes,
    )


def _bn_final_body(hin_ref, bnsum_ref, bnsq_ref, out_ref):
    n_total = 125 * _G
    tot = jnp.sum(bnsum_ref[...], axis=0)
    totsq = jnp.sum(bnsq_ref[...], axis=0)
    mu = tot / n_total
    var = totsq / n_total - mu * mu
    out_ref[0] = (hin_ref[0] - mu) * lax.rsqrt(var + 1e-5)


def kernel(x, edge_index, W0, as0, ad0, b0, W1, as1, ad1, b1,
           W2, as2, ad2, b2):
    counts = _build_counts(edge_index)

    h = x.reshape(_G, _NG0, _D)
    params = [(W0, as0, ad0, b0), (W1, as1, ad1, b1), (W2, as2, ad2, b2)]
    bnsum = bnsq = None
    n_g = _NG0
    c = counts
    for i in range(3):
        first = i == 0
        last = i == 2
        W, a_s, a_d, b = params[i]
        args = [h, c, W, a_s.reshape(1, _D), a_d.reshape(_D, 1),
                b.reshape(1, _D)]
        if not first:
            args += [bnsum, bnsq]
        outs = _make_layer(n_g, first, last)(*args)
        if last:
            h, bnsum, bnsq = outs
        else:
            h, c, bnsum, bnsq = outs
        n_g //= 2

    out = pl.pallas_call(
        _bn_final_body,
        grid=(_G,),
        in_specs=[
            pl.BlockSpec((1, 125, _D), lambda g: (g, 0, 0)),
            pl.BlockSpec((_G, 1, _D), lambda g: (0, 0, 0)),
            pl.BlockSpec((_G, 1, _D), lambda g: (0, 0, 0)),
        ],
        out_specs=pl.BlockSpec((1, 125, _D), lambda g: (g, 0, 0)),
        out_shape=jax.ShapeDtypeStruct((_G, 125, _D), jnp.float32),
    )(h, bnsum, bnsq)
    return out.reshape(_G, 125 * _D)
